# baseline (speedup 1.0000x reference)
import jax
import jax.numpy as jnp
from jax import lax
from jax.experimental import pallas as pl
from jax.experimental.pallas import tpu as pltpu, tpu_sc as plsc

AW = (10.0, 16.0, 33.0)
AH = (13.0, 30.0, 23.0)

HB = 4
GRID = 19

NJ = 17328
NJP = 17408
JQS = 4352
NH = 20


def _tc_body(x_ref, o_ref):
    i = pl.program_id(0)
    wof = jax.lax.broadcasted_iota(jnp.int32, (16, 76), 1).astype(jnp.float32)
    for hh in range(HB):
        hval = (i * HB + hh).astype(jnp.float32)
        for c in range(15):
            a, k = c // 5, c % 5
            v = x_ref[c, hh]
            if k in (0, 1, 4):
                s = jax.nn.sigmoid(v)
                if k == 0:
                    ov = (s + wof) * 8.0
                elif k == 1:
                    ov = (s + hval) * 8.0
                else:
                    ov = s
            else:
                ov = jnp.exp(v) * (AW[a] if k == 2 else AH[a])
            o_ref[hh, a, k] = ov


def _sc_body(x_ref, o_ref, in_v, out_v):
    wid = lax.axis_index("s") * 2 + lax.axis_index("c")
    iot = lax.iota(jnp.int32, 16)
    nrep = jnp.where(wid < 8, 2, 1)

    def unit(r, carry):
        u = wid + 32 * r
        k = u % 5
        bh = (u // 5) % 2
        jq = u // 10
        j0 = jq * JQS
        g0 = j0 // 3
        h0 = jnp.minimum(g0 // 76, 76 - NH)
        for a in range(3):
            pltpu.sync_copy(
                x_ref.at[a * 5 + k, pl.ds(h0, NH), pl.ds(8 * bh, 8), :],
                in_v.at[a])
        is_exp = jnp.logical_or(k == 2, k == 3)
        ancs = [jnp.where(k == 2, AW[a], AH[a]).astype(jnp.float32)
                for a in range(3)]

        def make_loop(compute_ov):
            def body(hh, c2):
                h = h0 + hh
                hof = h.astype(jnp.float32)
                jrow = 228 * h - j0
                for bb in range(8):
                    rowv = jnp.full((16,), bb, jnp.int32)
                    for w0 in (0, 16, 32, 48, 60):
                        jbase = jrow + 3 * w0 + 3 * iot
                        for a in range(3):
                            v = in_v[a, hh, bb, pl.ds(w0, 16)]
                            ov = compute_ov(v, a, w0, hof)
                            jl = jbase + a
                            m = jnp.logical_and(jl >= 0, jl < JQS)
                            plsc.store_scatter(out_v, [rowv, jl], ov, mask=m)
                return c2
            lax.fori_loop(0, NH, body, 0)

        @pl.when(is_exp)
        def _():
            make_loop(lambda v, a, w0, hof: jnp.exp(v) * ancs[a])

        @pl.when(jnp.logical_not(is_exp))
        def _():
            def ov_sig(v, a, w0, hof):
                s = 1.0 / (1.0 + jnp.exp(-v))
                wofv = (w0 + iot).astype(jnp.float32)
                return jnp.where(k == 0, (s + wofv) * 8.0,
                                 jnp.where(k == 1, (s + hof) * 8.0, s))
            make_loop(ov_sig)

        pltpu.sync_copy(out_v,
                        o_ref.at[k, pl.ds(8 * bh, 8), pl.ds(j0, JQS)])
        return carry

    lax.fori_loop(0, nrep, unit, 0)


def kernel(x, device, anchors_index):
    xt = jnp.transpose(x, (1, 2, 0, 3))
    o1 = pl.pallas_call(
        _tc_body,
        grid=(GRID,),
        in_specs=[pl.BlockSpec((15, HB, 16, 76), lambda i: (0, i, 0, 0))],
        out_specs=pl.BlockSpec((HB, 3, 5, 16, 76), lambda i: (i, 0, 0, 0, 0)),
        out_shape=jax.ShapeDtypeStruct((76, 3, 5, 16, 76), jnp.float32),
    )(xt)
    heat = jnp.transpose(o1, (3, 0, 4, 1, 2))

    mesh = plsc.VectorSubcoreMesh(core_axis_name="c", subcore_axis_name="s")
    o2 = pl.kernel(
        _sc_body,
        out_type=jax.ShapeDtypeStruct((5, 16, NJP), jnp.float32),
        mesh=mesh,
        scratch_types=[
            pltpu.VMEM((3, NH, 8, 76), jnp.float32),
            pltpu.VMEM((8, JQS), jnp.float32),
        ],
        compiler_params=pltpu.CompilerParams(needs_layout_passes=False),
    )(xt)
    xx = jnp.transpose(o2[:, :, :NJ], (1, 2, 0))
    return heat, xx
